# speedup vs baseline: 1.0010x; 1.0010x over previous
import jax
import jax.numpy as jnp
from jax.experimental import pallas as pl
from jax.experimental.pallas import tpu as pltpu


def _bn_scale_shift(s1, s2, n, g, be):
    mu = s1 / n
    var = s2 / n - mu * mu
    scale = g * jax.lax.rsqrt(var + 1e-5)
    shift = be - mu * scale
    return scale, shift


def _update_body(msum_ref, feat_ref, wu1_ref, bu1_ref, gu1_ref, beu1_ref,
                 wu2_ref, bu2_ref, gu2_ref, beu2_ref, out_ref):
    n = msum_ref.shape[0]
    inp2 = msum_ref[...] + feat_ref[...]
    pre1 = jnp.dot(inp2, wu1_ref[...], preferred_element_type=jnp.float32) + bu1_ref[...]
    s1 = jnp.sum(pre1, axis=0, keepdims=True)
    s2 = jnp.sum(pre1 * pre1, axis=0, keepdims=True)
    sc1, sh1 = _bn_scale_shift(s1, s2, n, gu1_ref[...], beu1_ref[...])
    hu = jnp.maximum(pre1 * sc1 + sh1, 0.0)
    pre2 = jnp.dot(hu, wu2_ref[...], preferred_element_type=jnp.float32) + bu2_ref[...]
    t1 = jnp.sum(pre2, axis=0, keepdims=True)
    t2 = jnp.sum(pre2 * pre2, axis=0, keepdims=True)
    sc2, sh2 = _bn_scale_shift(t1, t2, n, gu2_ref[...], beu2_ref[...])
    out_ref[...] = pre2 * sc2 + sh2 + feat_ref[...]


def kernel(x, feat, edge_index, W1, b1, g1, be1, W2, b2, g2, be2, Wse, bse,
           Wu1, bu1, gu1, beu1, Wu2, bu2, gu2, beu2):
    src = edge_index[0]
    dst = edge_index[1]
    n_nodes = feat.shape[0]
    sq = jnp.sum((x[src] - x[dst]) ** 2, axis=-1, keepdims=True)
    mi = jnp.concatenate([feat[src], feat[dst], sq], axis=-1)

    def _bn(h, g, b):
        mu = jnp.mean(h, axis=0, keepdims=True)
        var = jnp.var(h, axis=0, keepdims=True)
        return g * (h - mu) * jax.lax.rsqrt(var + 1e-5) + b

    h = jax.nn.relu(_bn(mi @ W1 + b1, g1, be1))
    msg = jax.nn.relu(_bn(h @ W2 + b2, g2, be2))
    ew = jax.nn.sigmoid(msg @ Wse + bse)
    m = msg * ew
    m_sum = jax.ops.segment_sum(m, dst, num_segments=n_nodes)

    out = pl.pallas_call(
        _update_body,
        out_shape=jax.ShapeDtypeStruct((n_nodes, 128), jnp.float32),
    )(m_sum, feat, Wu1, bu1.reshape(1, 128), gu1.reshape(1, 128),
      beu1.reshape(1, 128), Wu2, bu2.reshape(1, 128), gu2.reshape(1, 128),
      beu2.reshape(1, 128))
    return out


# speedup vs baseline: 2.8520x; 2.8492x over previous
import functools

import jax
import jax.numpy as jnp
from jax import lax
from jax.experimental import pallas as pl
from jax.experimental.pallas import tpu as pltpu
from jax.experimental.pallas import tpu_sc as plsc

N = 10000
E = 320000
H = 128

NC = 2
NS = 16
NW = NC * NS
EPW = E // NW
BE = 80
NB = EPW // BE
ZB = 200
NCH = N // ZB
DW = 256

def _k1_body(feat_ref, x_ref, w1a_ref, w1b_ref, b1_ref, t_ref, u_ref):
    f = feat_ref[...]
    xx = x_ref[...]
    zpad = jnp.zeros((f.shape[0], DW - H - 3), jnp.float32)
    p = jnp.dot(f, w1a_ref[...], preferred_element_type=jnp.float32)
    q = jnp.dot(f, w1b_ref[...], preferred_element_type=jnp.float32) + b1_ref[...]
    t_ref[...] = jnp.concatenate([p, xx, zpad], axis=1)
    u_ref[...] = jnp.concatenate([q, -xx, zpad], axis=1)


def _k2_gather_body(t_hbm, u_hbm, src_hbm, dst_hbm, s_hbm, sq_hbm,
                    idx_s, idx_d, bufP, bufQ, sqbuf, semP, semQ):
    wid = lax.axis_index("s") * NC + lax.axis_index("c")

    def body(it, carry):
        base = wid * EPW + it * BE
        pltpu.sync_copy(src_hbm.at[pl.ds(base, BE)], idx_s)
        pltpu.sync_copy(dst_hbm.at[pl.ds(base, BE)], idx_d)
        cpP = pltpu.async_copy(t_hbm.at[idx_s], bufP, semP)
        cpQ = pltpu.async_copy(u_hbm.at[idx_d], bufQ, semQ)
        cpP.wait()
        cpQ.wait()

        lanes = lax.iota(jnp.int32, 16)

        def grp(g, c):
            def inner(j2, acc):
                j = g * 16 + j2
                for k in range((H + 16) // 16):
                    ksl = pl.ds(k * 16, 16)
                    plsc.addupdate(bufP.at[j, ksl], bufQ[j, ksl])
                v = bufP[j, pl.ds(H, 16)]
                sq = v[0] * v[0] + v[1] * v[1] + v[2] * v[2]
                return jnp.where(lanes == j2, sq, acc)

            acc = lax.fori_loop(0, 16, inner, jnp.zeros((16,), jnp.float32))
            sqbuf[pl.ds(g * 16, 16)] = acc
            return c

        lax.fori_loop(0, BE // 16, grp, 0)
        pltpu.sync_copy(bufP.at[:, pl.ds(0, H)], s_hbm.at[pl.ds(base, BE)])
        pltpu.sync_copy(sqbuf, sq_hbm.at[pl.ds(base, BE)])
        return carry

    lax.fori_loop(0, NB, body, 0)


def _k3_body(s_ref, sq_ref, w1c_ref, st_ref):
    i = pl.program_id(0)
    pre1 = s_ref[...] + sq_ref[...] * w1c_ref[...]

    @pl.when(i == 0)
    def _():
        st_ref[...] = jnp.zeros_like(st_ref)

    st_ref[0:1, :] += jnp.sum(pre1, axis=0, keepdims=True)
    st_ref[1:2, :] += jnp.sum(pre1 * pre1, axis=0, keepdims=True)


def _k4_body(s_ref, sq_ref, w1c_ref, sc1_ref, sh1_ref, w2_ref, b2_ref,
             pre2_ref, st_ref):
    i = pl.program_id(0)
    pre1 = s_ref[...] + sq_ref[...] * w1c_ref[...]
    h = jnp.maximum(pre1 * sc1_ref[...] + sh1_ref[...], 0.0)
    pre2 = jnp.dot(h, w2_ref[...], preferred_element_type=jnp.float32) + b2_ref[...]
    pre2_ref[...] = pre2

    @pl.when(i == 0)
    def _():
        st_ref[...] = jnp.zeros_like(st_ref)

    st_ref[0:1, :] += jnp.sum(pre2, axis=0, keepdims=True)
    st_ref[1:2, :] += jnp.sum(pre2 * pre2, axis=0, keepdims=True)


def _k5_body(pre2_ref, sc2_ref, sh2_ref, wse_ref, bse_ref, m_ref):
    msg = jnp.maximum(pre2_ref[...] * sc2_ref[...] + sh2_ref[...], 0.0)
    s = jnp.sum(msg * wse_ref[...], axis=1, keepdims=True) + bse_ref[0, 0]
    ew = jax.nn.sigmoid(s)
    m_ref[...] = msg * ew


def _k6_scatter_body(m_hbm, dst_hbm, out_hbm, idx_d, mbuf, zbuf, acc):
    cid = lax.axis_index("c")
    sid = lax.axis_index("s")
    wid = sid * NC + cid

    def zrow(j, c):
        for k in range(H // 16):
            zbuf[j, pl.ds(k * 16, 16)] = jnp.zeros((16,), jnp.float32)
        return c

    lax.fori_loop(0, ZB, zrow, 0)
    for cc in range((NCH + NS - 1) // NS):
        ch = sid + NS * cc

        @pl.when(ch < NCH)
        def _():
            pltpu.sync_copy(zbuf, acc.at[pl.ds(ch * ZB, ZB)])

    plsc.subcore_barrier()

    def body(it, carry):
        base = wid * EPW + it * BE
        pltpu.sync_copy(dst_hbm.at[pl.ds(base, BE)], idx_d)
        pltpu.sync_copy(m_hbm.at[pl.ds(base, BE)], mbuf)
        pltpu.sync_copy(mbuf, acc.at[idx_d], add=True)
        return carry

    lax.fori_loop(0, NB, body, 0)
    plsc.subcore_barrier()
    for cc in range((NCH + NS - 1) // NS):
        ch = sid + NS * cc

        @pl.when(ch < NCH)
        def _():
            rows = pl.ds(ch * ZB, ZB)
            pltpu.sync_copy(acc.at[rows], out_hbm.at[cid, rows])


def _bn_scale_shift(s1, s2, n, g, be):
    mu = s1 / n
    var = s2 / n - mu * mu
    scale = g * lax.rsqrt(var + 1e-5)
    shift = be - mu * scale
    return scale, shift


@functools.cache
def _sc_kernels():
    mesh = plsc.VectorSubcoreMesh(core_axis_name="c", subcore_axis_name="s")
    k2 = functools.partial(
        pl.kernel,
        mesh=mesh,
        out_type=[
            jax.ShapeDtypeStruct((E, H), jnp.float32),
            jax.ShapeDtypeStruct((E,), jnp.float32),
        ],
        scratch_types=[
            pltpu.VMEM((BE,), jnp.int32),
            pltpu.VMEM((BE,), jnp.int32),
            pltpu.VMEM((BE, DW), jnp.float32),
            pltpu.VMEM((BE, DW), jnp.float32),
            pltpu.VMEM((BE,), jnp.float32),
            pltpu.SemaphoreType.DMA,
            pltpu.SemaphoreType.DMA,
        ],
    )(_k2_gather_body)
    k6 = functools.partial(
        pl.kernel,
        mesh=mesh,
        out_type=jax.ShapeDtypeStruct((NC, N, H), jnp.float32),
        scratch_types=[
            pltpu.VMEM((BE,), jnp.int32),
            pltpu.VMEM((BE, H), jnp.float32),
            pltpu.VMEM((ZB, H), jnp.float32),
            pltpu.VMEM_SHARED((N, H), jnp.float32),
        ],
    )(_k6_scatter_body)
    return k2, k6


def _k7_body(part_ref, feat_ref, wu1_ref, bu1_ref, gu1_ref, beu1_ref,
             wu2_ref, bu2_ref, gu2_ref, beu2_ref, out_ref):
    n = feat_ref.shape[0]
    feat = feat_ref[...]
    inp2 = part_ref[0] + part_ref[1] + feat
    pre1 = jnp.dot(inp2, wu1_ref[...], preferred_element_type=jnp.float32) + bu1_ref[...]
    s1 = jnp.sum(pre1, axis=0, keepdims=True)
    s2 = jnp.sum(pre1 * pre1, axis=0, keepdims=True)
    sc1, sh1 = _bn_scale_shift(s1, s2, n, gu1_ref[...], beu1_ref[...])
    hu = jnp.maximum(pre1 * sc1 + sh1, 0.0)
    pre2 = jnp.dot(hu, wu2_ref[...], preferred_element_type=jnp.float32) + bu2_ref[...]
    t1 = jnp.sum(pre2, axis=0, keepdims=True)
    t2 = jnp.sum(pre2 * pre2, axis=0, keepdims=True)
    sc2, sh2 = _bn_scale_shift(t1, t2, n, gu2_ref[...], beu2_ref[...])
    out_ref[...] = pre2 * sc2 + sh2 + feat


def kernel(x, feat, edge_index, W1, b1, g1, be1, W2, b2, g2, be2, Wse, bse,
           Wu1, bu1, gu1, beu1, Wu2, bu2, gu2, beu2):
    src = edge_index[0]
    dst = edge_index[1]
    w1a = W1[:H]
    w1b = W1[H:2 * H]
    w1c = W1[2 * H].reshape(1, H)
    t_tab, u_tab = pl.pallas_call(
        _k1_body,
        out_shape=[
            jax.ShapeDtypeStruct((N, DW), jnp.float32),
            jax.ShapeDtypeStruct((N, DW), jnp.float32),
        ],
    )(feat, x, w1a, w1b, b1.reshape(1, H))

    _k2_gather, _k6_scatter = _sc_kernels()
    s_arr, sq_arr = _k2_gather(t_tab, u_tab, src, dst)
    sq2 = sq_arr.reshape(E, 1)

    GB3 = 4000
    st1 = pl.pallas_call(
        _k3_body,
        grid=(E // GB3,),
        in_specs=[
            pl.BlockSpec((GB3, H), lambda i: (i, 0)),
            pl.BlockSpec((GB3, 1), lambda i: (i, 0)),
            pl.BlockSpec((1, H), lambda i: (0, 0)),
        ],
        out_specs=pl.BlockSpec((8, H), lambda i: (0, 0)),
        out_shape=jax.ShapeDtypeStruct((8, H), jnp.float32),
    )(s_arr, sq2, w1c)
    sc1, sh1 = _bn_scale_shift(st1[0:1], st1[1:2], E, g1.reshape(1, H),
                               be1.reshape(1, H))

    GB4 = 2000
    pre2, st2 = pl.pallas_call(
        _k4_body,
        grid=(E // GB4,),
        in_specs=[
            pl.BlockSpec((GB4, H), lambda i: (i, 0)),
            pl.BlockSpec((GB4, 1), lambda i: (i, 0)),
            pl.BlockSpec((1, H), lambda i: (0, 0)),
            pl.BlockSpec((1, H), lambda i: (0, 0)),
            pl.BlockSpec((1, H), lambda i: (0, 0)),
            pl.BlockSpec((H, H), lambda i: (0, 0)),
            pl.BlockSpec((1, H), lambda i: (0, 0)),
        ],
        out_specs=[
            pl.BlockSpec((GB4, H), lambda i: (i, 0)),
            pl.BlockSpec((8, H), lambda i: (0, 0)),
        ],
        out_shape=[
            jax.ShapeDtypeStruct((E, H), jnp.float32),
            jax.ShapeDtypeStruct((8, H), jnp.float32),
        ],
    )(s_arr, sq2, w1c, sc1, sh1, W2, b2.reshape(1, H))
    sc2, sh2 = _bn_scale_shift(st2[0:1], st2[1:2], E, g2.reshape(1, H),
                               be2.reshape(1, H))

    GB5 = 2000
    m_arr = pl.pallas_call(
        _k5_body,
        grid=(E // GB5,),
        in_specs=[
            pl.BlockSpec((GB5, H), lambda i: (i, 0)),
            pl.BlockSpec((1, H), lambda i: (0, 0)),
            pl.BlockSpec((1, H), lambda i: (0, 0)),
            pl.BlockSpec((1, H), lambda i: (0, 0)),
            pl.BlockSpec((1, 1), lambda i: (0, 0)),
        ],
        out_specs=pl.BlockSpec((GB5, H), lambda i: (i, 0)),
        out_shape=jax.ShapeDtypeStruct((E, H), jnp.float32),
    )(pre2, sc2, sh2, Wse.reshape(1, H), bse.reshape(1, 1))

    partials = _k6_scatter(m_arr, dst)

    out = pl.pallas_call(
        _k7_body,
        out_shape=jax.ShapeDtypeStruct((N, H), jnp.float32),
    )(partials, feat, Wu1, bu1.reshape(1, H), gu1.reshape(1, H),
      beu1.reshape(1, H), Wu2, bu2.reshape(1, H), gu2.reshape(1, H),
      beu2.reshape(1, H))
    return out


# speedup vs baseline: 3.4705x; 1.2169x over previous
import functools

import jax
import jax.numpy as jnp
from jax import lax
from jax.experimental import pallas as pl
from jax.experimental.pallas import tpu as pltpu
from jax.experimental.pallas import tpu_sc as plsc

N = 10000
E = 320000
H = 128

NC = 2
NS = 16
NW = NC * NS
EPW = E // NW
BE = 80
NB = EPW // BE
ZB = 200
NCH = N // ZB
DW = 256

def _k1_body(feat_ref, x_ref, w1a_ref, w1b_ref, b1_ref, t_ref, u_ref):
    f = feat_ref[...]
    xx = x_ref[...]
    zpad = jnp.zeros((f.shape[0], DW - H - 3), jnp.float32)
    p = jnp.dot(f, w1a_ref[...], preferred_element_type=jnp.float32)
    q = jnp.dot(f, w1b_ref[...], preferred_element_type=jnp.float32) + b1_ref[...]
    t_ref[...] = jnp.concatenate([p, xx, zpad], axis=1)
    u_ref[...] = jnp.concatenate([q, -xx, zpad], axis=1)


def _k2_gather_body(t_hbm, u_hbm, src_hbm, dst_hbm, s_hbm, sq_hbm,
                    idx_s0, idx_d0, idx_s1, idx_d1,
                    bufP0, bufQ0, bufP1, bufQ1, sq0, sq1,
                    semG0, semG1, semW0, semW1):
    wid = lax.axis_index("s") * NC + lax.axis_index("c")
    ebase = wid * EPW
    buf0 = (idx_s0, idx_d0, bufP0, bufQ0, sq0, semG0, semW0)
    buf1 = (idx_s1, idx_d1, bufP1, bufQ1, sq1, semG1, semW1)

    def fire_gather(blk, b):
        idx_s, idx_d, bufP, bufQ, _, semG, _ = b
        base = ebase + blk * BE
        pltpu.sync_copy(src_hbm.at[pl.ds(base, BE)], idx_s)
        pltpu.sync_copy(dst_hbm.at[pl.ds(base, BE)], idx_d)
        pltpu.async_copy(t_hbm.at[idx_s], bufP, semG)
        pltpu.async_copy(u_hbm.at[idx_d], bufQ, semG)

    def wait_gather(b):
        _, _, bufP, bufQ, _, semG, _ = b
        pltpu.make_async_copy(t_hbm.at[pl.ds(0, BE)], bufP, semG).wait()
        pltpu.make_async_copy(u_hbm.at[pl.ds(0, BE)], bufQ, semG).wait()

    def fire_wb(blk, b):
        _, _, bufP, _, sqb, _, semW = b
        base = ebase + blk * BE
        pltpu.async_copy(bufP.at[:, pl.ds(0, H)], s_hbm.at[pl.ds(base, BE)], semW)
        pltpu.async_copy(sqb, sq_hbm.at[pl.ds(base, BE)], semW)

    def wait_wb(b):
        _, _, bufP, _, sqb, _, semW = b
        pltpu.make_async_copy(bufP.at[:, pl.ds(0, H)],
                              s_hbm.at[pl.ds(0, BE)], semW).wait()
        pltpu.make_async_copy(sqb, sq_hbm.at[pl.ds(0, BE)], semW).wait()

    lanes = lax.iota(jnp.int32, 16)

    def compute(b):
        _, _, bufP, bufQ, sqb, _, _ = b

        def grp(g, c):
            def inner(j2, acc):
                j = g * 16 + j2
                for k in range((H + 16) // 16):
                    ksl = pl.ds(k * 16, 16)
                    plsc.addupdate(bufP.at[j, ksl], bufQ[j, ksl])
                v = bufP[j, pl.ds(H, 16)]
                sq = v[0] * v[0] + v[1] * v[1] + v[2] * v[2]
                return jnp.where(lanes == j2, sq, acc)

            acc = lax.fori_loop(0, 16, inner, jnp.zeros((16,), jnp.float32))
            sqb[pl.ds(g * 16, 16)] = acc
            return c

        lax.fori_loop(0, BE // 16, grp, 0)

    def stage(it, cur, nxt):
        wait_gather(cur)

        @pl.when(it >= 1)
        def _():
            wait_wb(nxt)

        @pl.when(it + 1 < NB)
        def _():
            fire_gather(it + 1, nxt)

        compute(cur)
        fire_wb(it, cur)

    fire_gather(0, buf0)

    def body(it, carry):
        @pl.when(it % 2 == 0)
        def _():
            stage(it, buf0, buf1)

        @pl.when(it % 2 == 1)
        def _():
            stage(it, buf1, buf0)

        return carry

    lax.fori_loop(0, NB, body, 0)
    wait_wb(buf0 if (NB - 1) % 2 == 0 else buf1)


def _k3_body(s_ref, sq_ref, w1c_ref, st_ref):
    i = pl.program_id(0)
    pre1 = s_ref[...] + sq_ref[...] * w1c_ref[...]

    @pl.when(i == 0)
    def _():
        st_ref[...] = jnp.zeros_like(st_ref)

    st_ref[0:1, :] += jnp.sum(pre1, axis=0, keepdims=True)
    st_ref[1:2, :] += jnp.sum(pre1 * pre1, axis=0, keepdims=True)


def _k4_body(s_ref, sq_ref, w1c_ref, sc1_ref, sh1_ref, w2_ref, b2_ref,
             pre2_ref, st_ref):
    i = pl.program_id(0)
    pre1 = s_ref[...] + sq_ref[...] * w1c_ref[...]
    h = jnp.maximum(pre1 * sc1_ref[...] + sh1_ref[...], 0.0)
    pre2 = jnp.dot(h, w2_ref[...], preferred_element_type=jnp.float32) + b2_ref[...]
    pre2_ref[...] = pre2

    @pl.when(i == 0)
    def _():
        st_ref[...] = jnp.zeros_like(st_ref)

    st_ref[0:1, :] += jnp.sum(pre2, axis=0, keepdims=True)
    st_ref[1:2, :] += jnp.sum(pre2 * pre2, axis=0, keepdims=True)


def _k5_body(pre2_ref, sc2_ref, sh2_ref, wse_ref, bse_ref, m_ref):
    msg = jnp.maximum(pre2_ref[...] * sc2_ref[...] + sh2_ref[...], 0.0)
    s = jnp.sum(msg * wse_ref[...], axis=1, keepdims=True) + bse_ref[0, 0]
    ew = jax.nn.sigmoid(s)
    m_ref[...] = msg * ew


def _k6_scatter_body(m_hbm, dst_hbm, out_hbm,
                     idx0, idx1, mb0, mb1, zbuf, acc,
                     semL0, semL1, semS0, semS1):
    cid = lax.axis_index("c")
    sid = lax.axis_index("s")
    wid = sid * NC + cid
    ebase = wid * EPW
    buf0 = (idx0, mb0, semL0, semS0)
    buf1 = (idx1, mb1, semL1, semS1)

    def zrow(j, c):
        for k in range(H // 16):
            zbuf[j, pl.ds(k * 16, 16)] = jnp.zeros((16,), jnp.float32)
        return c

    lax.fori_loop(0, ZB, zrow, 0)
    for cc in range((NCH + NS - 1) // NS):
        ch = sid + NS * cc

        @pl.when(ch < NCH)
        def _():
            pltpu.sync_copy(zbuf, acc.at[pl.ds(ch * ZB, ZB)])

    plsc.subcore_barrier()

    def fire_load(blk, b):
        idx, mb, semL, _ = b
        base = ebase + blk * BE
        pltpu.async_copy(dst_hbm.at[pl.ds(base, BE)], idx, semL)
        pltpu.async_copy(m_hbm.at[pl.ds(base, BE)], mb, semL)

    def wait_load(b):
        idx, mb, semL, _ = b
        pltpu.make_async_copy(dst_hbm.at[pl.ds(0, BE)], idx, semL).wait()
        pltpu.make_async_copy(m_hbm.at[pl.ds(0, BE)], mb, semL).wait()

    def fire_scatter(b):
        idx, mb, _, semS = b
        pltpu.async_copy(mb, acc.at[idx], semS, add=True)

    def wait_scatter(b):
        idx, mb, _, semS = b
        pltpu.make_async_copy(mb, acc.at[idx], semS).wait()

    def stage(it, cur, nxt):
        wait_load(cur)

        @pl.when(it >= 1)
        def _():
            wait_scatter(nxt)

        @pl.when(it + 1 < NB)
        def _():
            fire_load(it + 1, nxt)

        fire_scatter(cur)

    fire_load(0, buf0)

    def body(it, carry):
        @pl.when(it % 2 == 0)
        def _():
            stage(it, buf0, buf1)

        @pl.when(it % 2 == 1)
        def _():
            stage(it, buf1, buf0)

        return carry

    lax.fori_loop(0, NB, body, 0)
    wait_scatter(buf0 if (NB - 1) % 2 == 0 else buf1)
    plsc.subcore_barrier()
    for cc in range((NCH + NS - 1) // NS):
        ch = sid + NS * cc

        @pl.when(ch < NCH)
        def _():
            rows = pl.ds(ch * ZB, ZB)
            pltpu.sync_copy(acc.at[rows], out_hbm.at[cid, rows])


def _bn_scale_shift(s1, s2, n, g, be):
    mu = s1 / n
    var = s2 / n - mu * mu
    scale = g * lax.rsqrt(var + 1e-5)
    shift = be - mu * scale
    return scale, shift


@functools.cache
def _sc_kernels():
    mesh = plsc.VectorSubcoreMesh(core_axis_name="c", subcore_axis_name="s")
    k2 = functools.partial(
        pl.kernel,
        mesh=mesh,
        out_type=[
            jax.ShapeDtypeStruct((E, H), jnp.float32),
            jax.ShapeDtypeStruct((E,), jnp.float32),
        ],
        scratch_types=[
            pltpu.VMEM((BE,), jnp.int32),
            pltpu.VMEM((BE,), jnp.int32),
            pltpu.VMEM((BE,), jnp.int32),
            pltpu.VMEM((BE,), jnp.int32),
            pltpu.VMEM((BE, DW), jnp.float32),
            pltpu.VMEM((BE, DW), jnp.float32),
            pltpu.VMEM((BE, DW), jnp.float32),
            pltpu.VMEM((BE, DW), jnp.float32),
            pltpu.VMEM((BE,), jnp.float32),
            pltpu.VMEM((BE,), jnp.float32),
            pltpu.SemaphoreType.DMA,
            pltpu.SemaphoreType.DMA,
            pltpu.SemaphoreType.DMA,
            pltpu.SemaphoreType.DMA,
        ],
    )(_k2_gather_body)
    k6 = functools.partial(
        pl.kernel,
        mesh=mesh,
        out_type=jax.ShapeDtypeStruct((NC, N, H), jnp.float32),
        scratch_types=[
            pltpu.VMEM((BE,), jnp.int32),
            pltpu.VMEM((BE,), jnp.int32),
            pltpu.VMEM((BE, H), jnp.float32),
            pltpu.VMEM((BE, H), jnp.float32),
            pltpu.VMEM((ZB, H), jnp.float32),
            pltpu.VMEM_SHARED((N, H), jnp.float32),
            pltpu.SemaphoreType.DMA,
            pltpu.SemaphoreType.DMA,
            pltpu.SemaphoreType.DMA,
            pltpu.SemaphoreType.DMA,
        ],
    )(_k6_scatter_body)
    return k2, k6


def _k7_body(part_ref, feat_ref, wu1_ref, bu1_ref, gu1_ref, beu1_ref,
             wu2_ref, bu2_ref, gu2_ref, beu2_ref, out_ref):
    n = feat_ref.shape[0]
    feat = feat_ref[...]
    inp2 = part_ref[0] + part_ref[1] + feat
    pre1 = jnp.dot(inp2, wu1_ref[...], preferred_element_type=jnp.float32) + bu1_ref[...]
    s1 = jnp.sum(pre1, axis=0, keepdims=True)
    s2 = jnp.sum(pre1 * pre1, axis=0, keepdims=True)
    sc1, sh1 = _bn_scale_shift(s1, s2, n, gu1_ref[...], beu1_ref[...])
    hu = jnp.maximum(pre1 * sc1 + sh1, 0.0)
    pre2 = jnp.dot(hu, wu2_ref[...], preferred_element_type=jnp.float32) + bu2_ref[...]
    t1 = jnp.sum(pre2, axis=0, keepdims=True)
    t2 = jnp.sum(pre2 * pre2, axis=0, keepdims=True)
    sc2, sh2 = _bn_scale_shift(t1, t2, n, gu2_ref[...], beu2_ref[...])
    out_ref[...] = pre2 * sc2 + sh2 + feat


def kernel(x, feat, edge_index, W1, b1, g1, be1, W2, b2, g2, be2, Wse, bse,
           Wu1, bu1, gu1, beu1, Wu2, bu2, gu2, beu2):
    src = edge_index[0]
    dst = edge_index[1]
    w1a = W1[:H]
    w1b = W1[H:2 * H]
    w1c = W1[2 * H].reshape(1, H)
    t_tab, u_tab = pl.pallas_call(
        _k1_body,
        out_shape=[
            jax.ShapeDtypeStruct((N, DW), jnp.float32),
            jax.ShapeDtypeStruct((N, DW), jnp.float32),
        ],
    )(feat, x, w1a, w1b, b1.reshape(1, H))

    _k2_gather, _k6_scatter = _sc_kernels()
    s_arr, sq_arr = _k2_gather(t_tab, u_tab, src, dst)
    sq2 = sq_arr.reshape(E, 1)

    GB3 = 4000
    st1 = pl.pallas_call(
        _k3_body,
        grid=(E // GB3,),
        in_specs=[
            pl.BlockSpec((GB3, H), lambda i: (i, 0)),
            pl.BlockSpec((GB3, 1), lambda i: (i, 0)),
            pl.BlockSpec((1, H), lambda i: (0, 0)),
        ],
        out_specs=pl.BlockSpec((8, H), lambda i: (0, 0)),
        out_shape=jax.ShapeDtypeStruct((8, H), jnp.float32),
    )(s_arr, sq2, w1c)
    sc1, sh1 = _bn_scale_shift(st1[0:1], st1[1:2], E, g1.reshape(1, H),
                               be1.reshape(1, H))

    GB4 = 2000
    pre2, st2 = pl.pallas_call(
        _k4_body,
        grid=(E // GB4,),
        in_specs=[
            pl.BlockSpec((GB4, H), lambda i: (i, 0)),
            pl.BlockSpec((GB4, 1), lambda i: (i, 0)),
            pl.BlockSpec((1, H), lambda i: (0, 0)),
            pl.BlockSpec((1, H), lambda i: (0, 0)),
            pl.BlockSpec((1, H), lambda i: (0, 0)),
            pl.BlockSpec((H, H), lambda i: (0, 0)),
            pl.BlockSpec((1, H), lambda i: (0, 0)),
        ],
        out_specs=[
            pl.BlockSpec((GB4, H), lambda i: (i, 0)),
            pl.BlockSpec((8, H), lambda i: (0, 0)),
        ],
        out_shape=[
            jax.ShapeDtypeStruct((E, H), jnp.float32),
            jax.ShapeDtypeStruct((8, H), jnp.float32),
        ],
    )(s_arr, sq2, w1c, sc1, sh1, W2, b2.reshape(1, H))
    sc2, sh2 = _bn_scale_shift(st2[0:1], st2[1:2], E, g2.reshape(1, H),
                               be2.reshape(1, H))

    GB5 = 2000
    m_arr = pl.pallas_call(
        _k5_body,
        grid=(E // GB5,),
        in_specs=[
            pl.BlockSpec((GB5, H), lambda i: (i, 0)),
            pl.BlockSpec((1, H), lambda i: (0, 0)),
            pl.BlockSpec((1, H), lambda i: (0, 0)),
            pl.BlockSpec((1, H), lambda i: (0, 0)),
            pl.BlockSpec((1, 1), lambda i: (0, 0)),
        ],
        out_specs=pl.BlockSpec((GB5, H), lambda i: (i, 0)),
        out_shape=jax.ShapeDtypeStruct((E, H), jnp.float32),
    )(pre2, sc2, sh2, Wse.reshape(1, H), bse.reshape(1, 1))

    partials = _k6_scatter(m_arr, dst)

    out = pl.pallas_call(
        _k7_body,
        out_shape=jax.ShapeDtypeStruct((N, H), jnp.float32),
    )(partials, feat, Wu1, bu1.reshape(1, H), gu1.reshape(1, H),
      beu1.reshape(1, H), Wu2, bu2.reshape(1, H), gu2.reshape(1, H),
      beu2.reshape(1, H))
    return out


# speedup vs baseline: 3.8706x; 1.1153x over previous
import functools

import jax
import jax.numpy as jnp
from jax import lax
from jax.experimental import pallas as pl
from jax.experimental.pallas import tpu as pltpu
from jax.experimental.pallas import tpu_sc as plsc

N = 10000
E = 320000
H = 128

NC = 2
NS = 16
NW = NC * NS
EPW = E // NW
BE = 80
NB = EPW // BE
ZB = 200
NCH = N // ZB
DW = 256

def _k1_body(feat_ref, x_ref, w1a_ref, w1b_ref, b1_ref, t_ref, u_ref):
    f = feat_ref[...]
    xx = x_ref[...]
    zpad = jnp.zeros((f.shape[0], DW - H - 3), jnp.float32)
    p = jnp.dot(f, w1a_ref[...], preferred_element_type=jnp.float32)
    q = jnp.dot(f, w1b_ref[...], preferred_element_type=jnp.float32) + b1_ref[...]
    t_ref[...] = jnp.concatenate([p, xx, zpad], axis=1)
    u_ref[...] = jnp.concatenate([q, -xx, zpad], axis=1)


def _k2_gather_body(t_hbm, u_hbm, src_hbm, dst_hbm, s_hbm, sq_hbm,
                    idx_s0, idx_d0, idx_s1, idx_d1,
                    bufP0, bufQ0, bufP1, bufQ1, sq0, sq1,
                    semG0, semG1, semW0, semW1, semI0, semI1):
    wid = lax.axis_index("s") * NC + lax.axis_index("c")
    ebase = wid * EPW
    buf0 = (idx_s0, idx_d0, bufP0, bufQ0, sq0, semG0, semW0)
    buf1 = (idx_s1, idx_d1, bufP1, bufQ1, sq1, semG1, semW1)

    def fire_idx(blk, b, semI):
        idx_s, idx_d = b[0], b[1]
        base = ebase + blk * BE
        pltpu.async_copy(src_hbm.at[pl.ds(base, BE)], idx_s, semI)
        pltpu.async_copy(dst_hbm.at[pl.ds(base, BE)], idx_d, semI)

    def wait_idx(b, semI):
        idx_s, idx_d = b[0], b[1]
        pltpu.make_async_copy(src_hbm.at[pl.ds(0, BE)], idx_s, semI).wait()
        pltpu.make_async_copy(dst_hbm.at[pl.ds(0, BE)], idx_d, semI).wait()

    def fire_gather(b):
        idx_s, idx_d, bufP, bufQ, _, semG, _ = b
        pltpu.async_copy(t_hbm.at[idx_s], bufP, semG)
        pltpu.async_copy(u_hbm.at[idx_d], bufQ, semG)

    def wait_gather(b):
        _, _, bufP, bufQ, _, semG, _ = b
        pltpu.make_async_copy(t_hbm.at[pl.ds(0, BE)], bufP, semG).wait()
        pltpu.make_async_copy(u_hbm.at[pl.ds(0, BE)], bufQ, semG).wait()

    def fire_wb(blk, b):
        _, _, bufP, _, sqb, _, semW = b
        base = ebase + blk * BE
        pltpu.async_copy(bufP.at[:, pl.ds(0, H)], s_hbm.at[pl.ds(base, BE)], semW)
        pltpu.async_copy(sqb, sq_hbm.at[pl.ds(base, BE)], semW)

    def wait_wb(b):
        _, _, bufP, _, sqb, _, semW = b
        pltpu.make_async_copy(bufP.at[:, pl.ds(0, H)],
                              s_hbm.at[pl.ds(0, BE)], semW).wait()
        pltpu.make_async_copy(sqb, sq_hbm.at[pl.ds(0, BE)], semW).wait()

    lanes = lax.iota(jnp.int32, 16)

    def compute(b):
        _, _, bufP, bufQ, sqb, _, _ = b

        def grp(g, c):
            def inner(j2, acc):
                j = g * 16 + j2
                for k in range((H + 16) // 16):
                    ksl = pl.ds(k * 16, 16)
                    plsc.addupdate(bufP.at[j, ksl], bufQ[j, ksl])
                v = bufP[j, pl.ds(H, 16)]
                sq = v[0] * v[0] + v[1] * v[1] + v[2] * v[2]
                return jnp.where(lanes == j2, sq, acc)

            acc = lax.fori_loop(0, 16, inner, jnp.zeros((16,), jnp.float32))
            sqb[pl.ds(g * 16, 16)] = acc
            return c

        lax.fori_loop(0, BE // 16, grp, 0)

    def stage(it, cur, nxt, semIc, semIn):
        wait_gather(cur)

        @pl.when(it + 2 < NB)
        def _():
            fire_idx(it + 2, cur, semIc)

        @pl.when(it >= 1)
        def _():
            wait_wb(nxt)

        @pl.when(it + 1 < NB)
        def _():
            wait_idx(nxt, semIn)
            fire_gather(nxt)

        compute(cur)
        fire_wb(it, cur)

    pltpu.sync_copy(src_hbm.at[pl.ds(ebase, BE)], idx_s0)
    pltpu.sync_copy(dst_hbm.at[pl.ds(ebase, BE)], idx_d0)
    fire_gather(buf0)
    fire_idx(1, buf1, semI1)

    def body(it, carry):
        @pl.when(it % 2 == 0)
        def _():
            stage(it, buf0, buf1, semI0, semI1)

        @pl.when(it % 2 == 1)
        def _():
            stage(it, buf1, buf0, semI1, semI0)

        return carry

    lax.fori_loop(0, NB, body, 0)
    wait_wb(buf0 if (NB - 1) % 2 == 0 else buf1)


def _k3_body(s_ref, sq_ref, w1c_ref, st_ref):
    i = pl.program_id(0)
    pre1 = s_ref[...] + sq_ref[...] * w1c_ref[...]

    @pl.when(i == 0)
    def _():
        st_ref[...] = jnp.zeros_like(st_ref)

    st_ref[0:1, :] += jnp.sum(pre1, axis=0, keepdims=True)
    st_ref[1:2, :] += jnp.sum(pre1 * pre1, axis=0, keepdims=True)


def _k4_body(s_ref, sq_ref, w1c_ref, sc1_ref, sh1_ref, w2_ref, b2_ref,
             pre2_ref, st_ref):
    i = pl.program_id(0)
    pre1 = s_ref[...] + sq_ref[...] * w1c_ref[...]
    h = jnp.maximum(pre1 * sc1_ref[...] + sh1_ref[...], 0.0)
    pre2 = jnp.dot(h.astype(jnp.bfloat16), w2_ref[...].astype(jnp.bfloat16),
                   preferred_element_type=jnp.float32) + b2_ref[...]
    pre2_ref[...] = pre2.astype(jnp.bfloat16)

    @pl.when(i == 0)
    def _():
        st_ref[...] = jnp.zeros_like(st_ref)

    st_ref[0:1, :] += jnp.sum(pre2, axis=0, keepdims=True)
    st_ref[1:2, :] += jnp.sum(pre2 * pre2, axis=0, keepdims=True)


def _k5_body(pre2_ref, sc2_ref, sh2_ref, wse_ref, bse_ref, m_ref):
    msg = jnp.maximum(
        pre2_ref[...].astype(jnp.float32) * sc2_ref[...] + sh2_ref[...], 0.0)
    s = jnp.sum(msg * wse_ref[...], axis=1, keepdims=True) + bse_ref[0, 0]
    ew = jax.nn.sigmoid(s)
    m_ref[...] = msg * ew


def _k6_scatter_body(m_hbm, dst_hbm, out_hbm,
                     idx0, idx1, mb0, mb1, zbuf, acc,
                     semL0, semL1, semS0, semS1):
    cid = lax.axis_index("c")
    sid = lax.axis_index("s")
    wid = sid * NC + cid
    ebase = wid * EPW
    buf0 = (idx0, mb0, semL0, semS0)
    buf1 = (idx1, mb1, semL1, semS1)

    def zrow(j, c):
        for k in range(H // 16):
            zbuf[j, pl.ds(k * 16, 16)] = jnp.zeros((16,), jnp.float32)
        return c

    lax.fori_loop(0, ZB, zrow, 0)
    for cc in range((NCH + NS - 1) // NS):
        ch = sid + NS * cc

        @pl.when(ch < NCH)
        def _():
            pltpu.sync_copy(zbuf, acc.at[pl.ds(ch * ZB, ZB)])

    plsc.subcore_barrier()

    def fire_load(blk, b):
        idx, mb, semL, _ = b
        base = ebase + blk * BE
        pltpu.async_copy(dst_hbm.at[pl.ds(base, BE)], idx, semL)
        pltpu.async_copy(m_hbm.at[pl.ds(base, BE)], mb, semL)

    def wait_load(b):
        idx, mb, semL, _ = b
        pltpu.make_async_copy(dst_hbm.at[pl.ds(0, BE)], idx, semL).wait()
        pltpu.make_async_copy(m_hbm.at[pl.ds(0, BE)], mb, semL).wait()

    def fire_scatter(b):
        idx, mb, _, semS = b
        pltpu.async_copy(mb, acc.at[idx], semS, add=True)

    def wait_scatter(b):
        idx, mb, _, semS = b
        pltpu.make_async_copy(mb, acc.at[idx], semS).wait()

    def stage(it, cur, nxt):
        wait_load(cur)

        @pl.when(it >= 1)
        def _():
            wait_scatter(nxt)

        @pl.when(it + 1 < NB)
        def _():
            fire_load(it + 1, nxt)

        fire_scatter(cur)

    fire_load(0, buf0)

    def body(it, carry):
        @pl.when(it % 2 == 0)
        def _():
            stage(it, buf0, buf1)

        @pl.when(it % 2 == 1)
        def _():
            stage(it, buf1, buf0)

        return carry

    lax.fori_loop(0, NB, body, 0)
    wait_scatter(buf0 if (NB - 1) % 2 == 0 else buf1)
    plsc.subcore_barrier()
    for cc in range((NCH + NS - 1) // NS):
        ch = sid + NS * cc

        @pl.when(ch < NCH)
        def _():
            rows = pl.ds(ch * ZB, ZB)
            pltpu.sync_copy(acc.at[rows], out_hbm.at[cid, rows])


def _bn_scale_shift(s1, s2, n, g, be):
    mu = s1 / n
    var = s2 / n - mu * mu
    scale = g * lax.rsqrt(var + 1e-5)
    shift = be - mu * scale
    return scale, shift


@functools.cache
def _sc_kernels():
    mesh = plsc.VectorSubcoreMesh(core_axis_name="c", subcore_axis_name="s")
    k2 = functools.partial(
        pl.kernel,
        mesh=mesh,
        out_type=[
            jax.ShapeDtypeStruct((E, H), jnp.float32),
            jax.ShapeDtypeStruct((E,), jnp.float32),
        ],
        scratch_types=[
            pltpu.VMEM((BE,), jnp.int32),
            pltpu.VMEM((BE,), jnp.int32),
            pltpu.VMEM((BE,), jnp.int32),
            pltpu.VMEM((BE,), jnp.int32),
            pltpu.VMEM((BE, DW), jnp.float32),
            pltpu.VMEM((BE, DW), jnp.float32),
            pltpu.VMEM((BE, DW), jnp.float32),
            pltpu.VMEM((BE, DW), jnp.float32),
            pltpu.VMEM((BE,), jnp.float32),
            pltpu.VMEM((BE,), jnp.float32),
            pltpu.SemaphoreType.DMA,
            pltpu.SemaphoreType.DMA,
            pltpu.SemaphoreType.DMA,
            pltpu.SemaphoreType.DMA,
            pltpu.SemaphoreType.DMA,
            pltpu.SemaphoreType.DMA,
        ],
    )(_k2_gather_body)
    k6 = functools.partial(
        pl.kernel,
        mesh=mesh,
        out_type=jax.ShapeDtypeStruct((NC, N, H), jnp.float32),
        scratch_types=[
            pltpu.VMEM((BE,), jnp.int32),
            pltpu.VMEM((BE,), jnp.int32),
            pltpu.VMEM((BE, H), jnp.float32),
            pltpu.VMEM((BE, H), jnp.float32),
            pltpu.VMEM((ZB, H), jnp.float32),
            pltpu.VMEM_SHARED((N, H), jnp.float32),
            pltpu.SemaphoreType.DMA,
            pltpu.SemaphoreType.DMA,
            pltpu.SemaphoreType.DMA,
            pltpu.SemaphoreType.DMA,
        ],
    )(_k6_scatter_body)
    return k2, k6


def _k7_body(part_ref, feat_ref, wu1_ref, bu1_ref, gu1_ref, beu1_ref,
             wu2_ref, bu2_ref, gu2_ref, beu2_ref, out_ref):
    n = feat_ref.shape[0]
    feat = feat_ref[...]
    inp2 = part_ref[0] + part_ref[1] + feat
    pre1 = jnp.dot(inp2, wu1_ref[...], preferred_element_type=jnp.float32) + bu1_ref[...]
    s1 = jnp.sum(pre1, axis=0, keepdims=True)
    s2 = jnp.sum(pre1 * pre1, axis=0, keepdims=True)
    sc1, sh1 = _bn_scale_shift(s1, s2, n, gu1_ref[...], beu1_ref[...])
    hu = jnp.maximum(pre1 * sc1 + sh1, 0.0)
    pre2 = jnp.dot(hu, wu2_ref[...], preferred_element_type=jnp.float32) + bu2_ref[...]
    t1 = jnp.sum(pre2, axis=0, keepdims=True)
    t2 = jnp.sum(pre2 * pre2, axis=0, keepdims=True)
    sc2, sh2 = _bn_scale_shift(t1, t2, n, gu2_ref[...], beu2_ref[...])
    out_ref[...] = pre2 * sc2 + sh2 + feat


def kernel(x, feat, edge_index, W1, b1, g1, be1, W2, b2, g2, be2, Wse, bse,
           Wu1, bu1, gu1, beu1, Wu2, bu2, gu2, beu2):
    src = edge_index[0]
    dst = edge_index[1]
    w1a = W1[:H]
    w1b = W1[H:2 * H]
    w1c = W1[2 * H].reshape(1, H)
    t_tab, u_tab = pl.pallas_call(
        _k1_body,
        out_shape=[
            jax.ShapeDtypeStruct((N, DW), jnp.float32),
            jax.ShapeDtypeStruct((N, DW), jnp.float32),
        ],
    )(feat, x, w1a, w1b, b1.reshape(1, H))

    _k2_gather, _k6_scatter = _sc_kernels()
    s_arr, sq_arr = _k2_gather(t_tab, u_tab, src, dst)
    sq2 = sq_arr.reshape(E, 1)

    GB3 = 4000
    st1 = pl.pallas_call(
        _k3_body,
        grid=(E // GB3,),
        in_specs=[
            pl.BlockSpec((GB3, H), lambda i: (i, 0)),
            pl.BlockSpec((GB3, 1), lambda i: (i, 0)),
            pl.BlockSpec((1, H), lambda i: (0, 0)),
        ],
        out_specs=pl.BlockSpec((8, H), lambda i: (0, 0)),
        out_shape=jax.ShapeDtypeStruct((8, H), jnp.float32),
    )(s_arr, sq2, w1c)
    sc1, sh1 = _bn_scale_shift(st1[0:1], st1[1:2], E, g1.reshape(1, H),
                               be1.reshape(1, H))

    GB4 = 2000
    pre2, st2 = pl.pallas_call(
        _k4_body,
        grid=(E // GB4,),
        in_specs=[
            pl.BlockSpec((GB4, H), lambda i: (i, 0)),
            pl.BlockSpec((GB4, 1), lambda i: (i, 0)),
            pl.BlockSpec((1, H), lambda i: (0, 0)),
            pl.BlockSpec((1, H), lambda i: (0, 0)),
            pl.BlockSpec((1, H), lambda i: (0, 0)),
            pl.BlockSpec((H, H), lambda i: (0, 0)),
            pl.BlockSpec((1, H), lambda i: (0, 0)),
        ],
        out_specs=[
            pl.BlockSpec((GB4, H), lambda i: (i, 0)),
            pl.BlockSpec((8, H), lambda i: (0, 0)),
        ],
        out_shape=[
            jax.ShapeDtypeStruct((E, H), jnp.bfloat16),
            jax.ShapeDtypeStruct((8, H), jnp.float32),
        ],
    )(s_arr, sq2, w1c, sc1, sh1, W2, b2.reshape(1, H))
    sc2, sh2 = _bn_scale_shift(st2[0:1], st2[1:2], E, g2.reshape(1, H),
                               be2.reshape(1, H))

    GB5 = 2000
    m_arr = pl.pallas_call(
        _k5_body,
        grid=(E // GB5,),
        in_specs=[
            pl.BlockSpec((GB5, H), lambda i: (i, 0)),
            pl.BlockSpec((1, H), lambda i: (0, 0)),
            pl.BlockSpec((1, H), lambda i: (0, 0)),
            pl.BlockSpec((1, H), lambda i: (0, 0)),
            pl.BlockSpec((1, 1), lambda i: (0, 0)),
        ],
        out_specs=pl.BlockSpec((GB5, H), lambda i: (i, 0)),
        out_shape=jax.ShapeDtypeStruct((E, H), jnp.float32),
    )(pre2, sc2, sh2, Wse.reshape(1, H), bse.reshape(1, 1))

    partials = _k6_scatter(m_arr, dst)

    out = pl.pallas_call(
        _k7_body,
        out_shape=jax.ShapeDtypeStruct((N, H), jnp.float32),
    )(partials, feat, Wu1, bu1.reshape(1, H), gu1.reshape(1, H),
      beu1.reshape(1, H), Wu2, bu2.reshape(1, H), gu2.reshape(1, H),
      beu2.reshape(1, H))
    return out


# speedup vs baseline: 5.1646x; 1.3343x over previous
import functools

import jax
import jax.numpy as jnp
from jax import lax
from jax.experimental import pallas as pl
from jax.experimental.pallas import tpu as pltpu
from jax.experimental.pallas import tpu_sc as plsc

N = 10000
E = 320000
H = 128

NC = 2
NS = 16
NW = NC * NS
EPW = E // NW
BE = 80
NB = EPW // BE
ZB = 200
NCH = N // ZB
DW = 256

def _k1_body(feat_ref, x_ref, w1a_ref, w1b_ref, b1_ref, t_ref, u_ref):
    f = feat_ref[...]
    xx = x_ref[...]
    zpad = jnp.zeros((f.shape[0], DW - H - 3), jnp.float32)
    p = jnp.dot(f, w1a_ref[...], preferred_element_type=jnp.float32)
    q = jnp.dot(f, w1b_ref[...], preferred_element_type=jnp.float32) + b1_ref[...]
    t_ref[...] = jnp.concatenate([p, xx, zpad], axis=1)
    u_ref[...] = jnp.concatenate([q, -xx, zpad], axis=1)


def _k2_gather_body(t_hbm, u_hbm, w1c_hbm, src_hbm, dst_hbm, s_hbm,
                    idx_s0, idx_d0, idx_s1, idx_d1,
                    bufP0, bufQ0, bufP1, bufQ1, w1cv,
                    semG0, semG1, semW0, semW1, semI0, semI1):
    wid = lax.axis_index("s") * NC + lax.axis_index("c")
    ebase = wid * EPW
    buf0 = (idx_s0, idx_d0, bufP0, bufQ0, semG0, semW0)
    buf1 = (idx_s1, idx_d1, bufP1, bufQ1, semG1, semW1)
    pltpu.sync_copy(w1c_hbm, w1cv)
    w1c_regs = [w1cv[pl.ds(k * 16, 16)] for k in range(H // 16)]

    def fire_idx(blk, b, semI):
        idx_s, idx_d = b[0], b[1]
        base = ebase + blk * BE
        pltpu.async_copy(src_hbm.at[pl.ds(base, BE)], idx_s, semI)
        pltpu.async_copy(dst_hbm.at[pl.ds(base, BE)], idx_d, semI)

    def wait_idx(b, semI):
        idx_s, idx_d = b[0], b[1]
        pltpu.make_async_copy(src_hbm.at[pl.ds(0, BE)], idx_s, semI).wait()
        pltpu.make_async_copy(dst_hbm.at[pl.ds(0, BE)], idx_d, semI).wait()

    def fire_gather(b):
        idx_s, idx_d, bufP, bufQ, semG, _ = b
        pltpu.async_copy(t_hbm.at[idx_s], bufP, semG)
        pltpu.async_copy(u_hbm.at[idx_d], bufQ, semG)

    def wait_gather(b):
        _, _, bufP, bufQ, semG, _ = b
        pltpu.make_async_copy(t_hbm.at[pl.ds(0, BE)], bufP, semG).wait()
        pltpu.make_async_copy(u_hbm.at[pl.ds(0, BE)], bufQ, semG).wait()

    def fire_wb(blk, b):
        _, _, bufP, _, _, semW = b
        base = ebase + blk * BE
        pltpu.async_copy(bufP.at[:, pl.ds(0, H)], s_hbm.at[pl.ds(base, BE)], semW)

    def wait_wb(b):
        _, _, bufP, _, _, semW = b
        pltpu.make_async_copy(bufP.at[:, pl.ds(0, H)],
                              s_hbm.at[pl.ds(0, BE)], semW).wait()

    def compute(b):
        _, _, bufP, bufQ, _, _ = b

        def row(j, c):
            vx = bufP[j, pl.ds(H, 16)] + bufQ[j, pl.ds(H, 16)]
            sq = vx[0] * vx[0] + vx[1] * vx[1] + vx[2] * vx[2]
            for k in range(H // 16):
                ksl = pl.ds(k * 16, 16)
                bufP[j, ksl] = bufP[j, ksl] + bufQ[j, ksl] + sq * w1c_regs[k]
            return c

        lax.fori_loop(0, BE, row, 0)

    def stage(it, cur, nxt, semIc, semIn):
        wait_gather(cur)

        @pl.when(it + 2 < NB)
        def _():
            fire_idx(it + 2, cur, semIc)

        @pl.when(it >= 1)
        def _():
            wait_wb(nxt)

        @pl.when(it + 1 < NB)
        def _():
            wait_idx(nxt, semIn)
            fire_gather(nxt)

        compute(cur)
        fire_wb(it, cur)

    pltpu.sync_copy(src_hbm.at[pl.ds(ebase, BE)], idx_s0)
    pltpu.sync_copy(dst_hbm.at[pl.ds(ebase, BE)], idx_d0)
    fire_gather(buf0)
    fire_idx(1, buf1, semI1)

    def body(it, carry):
        @pl.when(it % 2 == 0)
        def _():
            stage(it, buf0, buf1, semI0, semI1)

        @pl.when(it % 2 == 1)
        def _():
            stage(it, buf1, buf0, semI1, semI0)

        return carry

    lax.fori_loop(0, NB, body, 0)
    wait_wb(buf0 if (NB - 1) % 2 == 0 else buf1)


def _k3_body(s_ref, st_ref):
    i = pl.program_id(0)
    pre1 = s_ref[...]

    @pl.when(i == 0)
    def _():
        st_ref[...] = jnp.zeros_like(st_ref)

    st_ref[0:1, :] += jnp.sum(pre1, axis=0, keepdims=True)
    st_ref[1:2, :] += jnp.sum(pre1 * pre1, axis=0, keepdims=True)


def _k4_body(s_ref, sc1_ref, sh1_ref, w2_ref, b2_ref,
             pre2_ref, st_ref):
    i = pl.program_id(0)
    h = jnp.maximum(s_ref[...] * sc1_ref[...] + sh1_ref[...], 0.0)
    pre2 = jnp.dot(h.astype(jnp.bfloat16), w2_ref[...].astype(jnp.bfloat16),
                   preferred_element_type=jnp.float32) + b2_ref[...]
    pre2_ref[...] = pre2.astype(jnp.bfloat16)

    @pl.when(i == 0)
    def _():
        st_ref[...] = jnp.zeros_like(st_ref)

    st_ref[0:1, :] += jnp.sum(pre2, axis=0, keepdims=True)
    st_ref[1:2, :] += jnp.sum(pre2 * pre2, axis=0, keepdims=True)


def _k5_body(pre2_ref, sc2_ref, sh2_ref, wse_ref, bse_ref, m_ref):
    msg = jnp.maximum(
        pre2_ref[...].astype(jnp.float32) * sc2_ref[...] + sh2_ref[...], 0.0)
    s = jnp.sum(msg * wse_ref[...], axis=1, keepdims=True) + bse_ref[0, 0]
    ew = jax.nn.sigmoid(s)
    m_ref[...] = msg * ew


def _k6_scatter_body(m_hbm, dst_hbm, out_hbm,
                     idx0, idx1, mb0, mb1, zbuf, acc,
                     semL0, semL1, semS0, semS1):
    cid = lax.axis_index("c")
    sid = lax.axis_index("s")
    wid = sid * NC + cid
    ebase = wid * EPW
    buf0 = (idx0, mb0, semL0, semS0)
    buf1 = (idx1, mb1, semL1, semS1)

    def zrow(j, c):
        for k in range(H // 16):
            zbuf[j, pl.ds(k * 16, 16)] = jnp.zeros((16,), jnp.float32)
        return c

    lax.fori_loop(0, ZB, zrow, 0)
    for cc in range((NCH + NS - 1) // NS):
        ch = sid + NS * cc

        @pl.when(ch < NCH)
        def _():
            pltpu.sync_copy(zbuf, acc.at[pl.ds(ch * ZB, ZB)])

    plsc.subcore_barrier()

    def fire_load(blk, b):
        idx, mb, semL, _ = b
        base = ebase + blk * BE
        pltpu.async_copy(dst_hbm.at[pl.ds(base, BE)], idx, semL)
        pltpu.async_copy(m_hbm.at[pl.ds(base, BE)], mb, semL)

    def wait_load(b):
        idx, mb, semL, _ = b
        pltpu.make_async_copy(dst_hbm.at[pl.ds(0, BE)], idx, semL).wait()
        pltpu.make_async_copy(m_hbm.at[pl.ds(0, BE)], mb, semL).wait()

    def fire_scatter(b):
        idx, mb, _, semS = b
        pltpu.async_copy(mb, acc.at[idx], semS, add=True)

    def wait_scatter(b):
        idx, mb, _, semS = b
        pltpu.make_async_copy(mb, acc.at[idx], semS).wait()

    def stage(it, cur, nxt):
        wait_load(cur)

        @pl.when(it >= 1)
        def _():
            wait_scatter(nxt)

        @pl.when(it + 1 < NB)
        def _():
            fire_load(it + 1, nxt)

        fire_scatter(cur)

    fire_load(0, buf0)

    def body(it, carry):
        @pl.when(it % 2 == 0)
        def _():
            stage(it, buf0, buf1)

        @pl.when(it % 2 == 1)
        def _():
            stage(it, buf1, buf0)

        return carry

    lax.fori_loop(0, NB, body, 0)
    wait_scatter(buf0 if (NB - 1) % 2 == 0 else buf1)
    plsc.subcore_barrier()
    for cc in range((NCH + NS - 1) // NS):
        ch = sid + NS * cc

        @pl.when(ch < NCH)
        def _():
            rows = pl.ds(ch * ZB, ZB)
            pltpu.sync_copy(acc.at[rows], out_hbm.at[cid, rows])


def _bn_scale_shift(s1, s2, n, g, be):
    mu = s1 / n
    var = s2 / n - mu * mu
    scale = g * lax.rsqrt(var + 1e-5)
    shift = be - mu * scale
    return scale, shift


@functools.cache
def _sc_kernels():
    mesh = plsc.VectorSubcoreMesh(core_axis_name="c", subcore_axis_name="s")
    k2 = functools.partial(
        pl.kernel,
        mesh=mesh,
        out_type=jax.ShapeDtypeStruct((E, H), jnp.float32),
        scratch_types=[
            pltpu.VMEM((BE,), jnp.int32),
            pltpu.VMEM((BE,), jnp.int32),
            pltpu.VMEM((BE,), jnp.int32),
            pltpu.VMEM((BE,), jnp.int32),
            pltpu.VMEM((BE, DW), jnp.float32),
            pltpu.VMEM((BE, DW), jnp.float32),
            pltpu.VMEM((BE, DW), jnp.float32),
            pltpu.VMEM((BE, DW), jnp.float32),
            pltpu.VMEM((H,), jnp.float32),
            pltpu.SemaphoreType.DMA,
            pltpu.SemaphoreType.DMA,
            pltpu.SemaphoreType.DMA,
            pltpu.SemaphoreType.DMA,
            pltpu.SemaphoreType.DMA,
            pltpu.SemaphoreType.DMA,
        ],
    )(_k2_gather_body)
    k6 = functools.partial(
        pl.kernel,
        mesh=mesh,
        out_type=jax.ShapeDtypeStruct((NC, N, H), jnp.float32),
        scratch_types=[
            pltpu.VMEM((BE,), jnp.int32),
            pltpu.VMEM((BE,), jnp.int32),
            pltpu.VMEM((BE, H), jnp.float32),
            pltpu.VMEM((BE, H), jnp.float32),
            pltpu.VMEM((ZB, H), jnp.float32),
            pltpu.VMEM_SHARED((N, H), jnp.float32),
            pltpu.SemaphoreType.DMA,
            pltpu.SemaphoreType.DMA,
            pltpu.SemaphoreType.DMA,
            pltpu.SemaphoreType.DMA,
        ],
    )(_k6_scatter_body)
    return k2, k6


def _k7_body(part_ref, feat_ref, wu1_ref, bu1_ref, gu1_ref, beu1_ref,
             wu2_ref, bu2_ref, gu2_ref, beu2_ref, out_ref):
    n = feat_ref.shape[0]
    feat = feat_ref[...]
    inp2 = part_ref[0] + part_ref[1] + feat
    pre1 = jnp.dot(inp2, wu1_ref[...], preferred_element_type=jnp.float32) + bu1_ref[...]
    s1 = jnp.sum(pre1, axis=0, keepdims=True)
    s2 = jnp.sum(pre1 * pre1, axis=0, keepdims=True)
    sc1, sh1 = _bn_scale_shift(s1, s2, n, gu1_ref[...], beu1_ref[...])
    hu = jnp.maximum(pre1 * sc1 + sh1, 0.0)
    pre2 = jnp.dot(hu, wu2_ref[...], preferred_element_type=jnp.float32) + bu2_ref[...]
    t1 = jnp.sum(pre2, axis=0, keepdims=True)
    t2 = jnp.sum(pre2 * pre2, axis=0, keepdims=True)
    sc2, sh2 = _bn_scale_shift(t1, t2, n, gu2_ref[...], beu2_ref[...])
    out_ref[...] = pre2 * sc2 + sh2 + feat


def kernel(x, feat, edge_index, W1, b1, g1, be1, W2, b2, g2, be2, Wse, bse,
           Wu1, bu1, gu1, beu1, Wu2, bu2, gu2, beu2):
    src = edge_index[0]
    dst = edge_index[1]
    w1a = W1[:H]
    w1b = W1[H:2 * H]
    w1c = W1[2 * H].reshape(1, H)
    t_tab, u_tab = pl.pallas_call(
        _k1_body,
        out_shape=[
            jax.ShapeDtypeStruct((N, DW), jnp.float32),
            jax.ShapeDtypeStruct((N, DW), jnp.float32),
        ],
    )(feat, x, w1a, w1b, b1.reshape(1, H))

    _k2_gather, _k6_scatter = _sc_kernels()
    s_arr = _k2_gather(t_tab, u_tab, w1c.reshape(H), src, dst)

    GB3 = 4000
    st1 = pl.pallas_call(
        _k3_body,
        grid=(E // GB3,),
        in_specs=[
            pl.BlockSpec((GB3, H), lambda i: (i, 0)),
        ],
        out_specs=pl.BlockSpec((8, H), lambda i: (0, 0)),
        out_shape=jax.ShapeDtypeStruct((8, H), jnp.float32),
    )(s_arr)
    sc1, sh1 = _bn_scale_shift(st1[0:1], st1[1:2], E, g1.reshape(1, H),
                               be1.reshape(1, H))

    GB4 = 4000
    pre2, st2 = pl.pallas_call(
        _k4_body,
        grid=(E // GB4,),
        in_specs=[
            pl.BlockSpec((GB4, H), lambda i: (i, 0)),
            pl.BlockSpec((1, H), lambda i: (0, 0)),
            pl.BlockSpec((1, H), lambda i: (0, 0)),
            pl.BlockSpec((H, H), lambda i: (0, 0)),
            pl.BlockSpec((1, H), lambda i: (0, 0)),
        ],
        out_specs=[
            pl.BlockSpec((GB4, H), lambda i: (i, 0)),
            pl.BlockSpec((8, H), lambda i: (0, 0)),
        ],
        out_shape=[
            jax.ShapeDtypeStruct((E, H), jnp.bfloat16),
            jax.ShapeDtypeStruct((8, H), jnp.float32),
        ],
    )(s_arr, sc1, sh1, W2, b2.reshape(1, H))
    sc2, sh2 = _bn_scale_shift(st2[0:1], st2[1:2], E, g2.reshape(1, H),
                               be2.reshape(1, H))

    GB5 = 4000
    m_arr = pl.pallas_call(
        _k5_body,
        grid=(E // GB5,),
        in_specs=[
            pl.BlockSpec((GB5, H), lambda i: (i, 0)),
            pl.BlockSpec((1, H), lambda i: (0, 0)),
            pl.BlockSpec((1, H), lambda i: (0, 0)),
            pl.BlockSpec((1, H), lambda i: (0, 0)),
            pl.BlockSpec((1, 1), lambda i: (0, 0)),
        ],
        out_specs=pl.BlockSpec((GB5, H), lambda i: (i, 0)),
        out_shape=jax.ShapeDtypeStruct((E, H), jnp.float32),
    )(pre2, sc2, sh2, Wse.reshape(1, H), bse.reshape(1, 1))

    partials = _k6_scatter(m_arr, dst)

    out = pl.pallas_call(
        _k7_body,
        out_shape=jax.ShapeDtypeStruct((N, H), jnp.float32),
    )(partials, feat, Wu1, bu1.reshape(1, H), gu1.reshape(1, H),
      beu1.reshape(1, H), Wu2, bu2.reshape(1, H), gu2.reshape(1, H),
      beu2.reshape(1, H))
    return out


# speedup vs baseline: 5.8000x; 1.1230x over previous
import functools

import jax
import jax.numpy as jnp
from jax import lax
from jax.experimental import pallas as pl
from jax.experimental.pallas import tpu as pltpu
from jax.experimental.pallas import tpu_sc as plsc

N = 10000
E = 320000
H = 128

NC = 2
NS = 16
NW = NC * NS
EPW = E // NW
BE = 80
NB = EPW // BE
ZB = 200
NCH = N // ZB
DW = 256

def _k1_body(feat_ref, x_ref, w1a_ref, w1b_ref, b1_ref, t_ref, u_ref):
    f = feat_ref[...]
    xx = x_ref[...]
    zpad = jnp.zeros((f.shape[0], DW - H - 3), jnp.float32)
    p = jnp.dot(f, w1a_ref[...], preferred_element_type=jnp.float32)
    q = jnp.dot(f, w1b_ref[...], preferred_element_type=jnp.float32) + b1_ref[...]
    t_ref[...] = jnp.concatenate([p, xx, zpad], axis=1)
    u_ref[...] = jnp.concatenate([q, -xx, zpad], axis=1)


def _k2_gather_body(t_hbm, u_hbm, w1c_hbm, src_hbm, dst_hbm, s_hbm, mom_hbm,
                    idx_s0, idx_d0, idx_s1, idx_d1,
                    bufP0, bufQ0, bufP1, bufQ1, w1cv, momv,
                    semG0, semG1, semW0, semW1, semI0, semI1):
    wid = lax.axis_index("s") * NC + lax.axis_index("c")
    ebase = wid * EPW
    buf0 = (idx_s0, idx_d0, bufP0, bufQ0, semG0, semW0)
    buf1 = (idx_s1, idx_d1, bufP1, bufQ1, semG1, semW1)
    pltpu.sync_copy(w1c_hbm, w1cv)
    w1c_regs = [w1cv[pl.ds(k * 16, 16)] for k in range(H // 16)]

    def fire_idx(blk, b, semI):
        idx_s, idx_d = b[0], b[1]
        base = ebase + blk * BE
        pltpu.async_copy(src_hbm.at[pl.ds(base, BE)], idx_s, semI)
        pltpu.async_copy(dst_hbm.at[pl.ds(base, BE)], idx_d, semI)

    def wait_idx(b, semI):
        idx_s, idx_d = b[0], b[1]
        pltpu.make_async_copy(src_hbm.at[pl.ds(0, BE)], idx_s, semI).wait()
        pltpu.make_async_copy(dst_hbm.at[pl.ds(0, BE)], idx_d, semI).wait()

    def fire_gather(b):
        idx_s, idx_d, bufP, bufQ, semG, _ = b
        pltpu.async_copy(t_hbm.at[idx_s], bufP, semG)
        pltpu.async_copy(u_hbm.at[idx_d], bufQ, semG)

    def wait_gather(b):
        idx_s, idx_d, bufP, bufQ, semG, _ = b
        pltpu.make_async_copy(t_hbm.at[idx_s], bufP, semG).wait()
        pltpu.make_async_copy(u_hbm.at[idx_d], bufQ, semG).wait()

    def fire_wb(blk, b):
        _, _, bufP, _, _, semW = b
        base = ebase + blk * BE
        pltpu.async_copy(bufP.at[:, pl.ds(0, H)], s_hbm.at[pl.ds(base, BE)], semW)

    def wait_wb(b):
        _, _, bufP, _, _, semW = b
        pltpu.make_async_copy(bufP.at[:, pl.ds(0, H)],
                              s_hbm.at[pl.ds(0, BE)], semW).wait()

    def compute(b, mom):
        _, _, bufP, bufQ, _, _ = b

        def row(j, mm):
            m1, m2 = mm
            vx = bufP[j, pl.ds(H, 16)] + bufQ[j, pl.ds(H, 16)]
            sq = vx[0] * vx[0] + vx[1] * vx[1] + vx[2] * vx[2]
            n1, n2 = [], []
            for k in range(H // 16):
                ksl = pl.ds(k * 16, 16)
                v = bufP[j, ksl] + bufQ[j, ksl] + sq * w1c_regs[k]
                bufP[j, ksl] = v
                n1.append(m1[k] + v)
                n2.append(m2[k] + v * v)
            return (tuple(n1), tuple(n2))

        return lax.fori_loop(0, BE, row, mom)

    def stage(it, cur, nxt, semIc, semIn, mom):
        wait_gather(cur)

        @pl.when(it + 2 < NB)
        def _():
            fire_idx(it + 2, cur, semIc)

        @pl.when(it >= 1)
        def _():
            wait_wb(nxt)

        @pl.when(it + 1 < NB)
        def _():
            wait_idx(nxt, semIn)
            fire_gather(nxt)

        mom = compute(cur, mom)
        fire_wb(it, cur)
        return mom

    pltpu.sync_copy(src_hbm.at[pl.ds(ebase, BE)], idx_s0)
    pltpu.sync_copy(dst_hbm.at[pl.ds(ebase, BE)], idx_d0)
    fire_gather(buf0)
    fire_idx(1, buf1, semI1)

    zv = jnp.zeros((16,), jnp.float32)
    mom0 = (tuple(zv for _ in range(H // 16)), tuple(zv for _ in range(H // 16)))

    def body(i2, mom):
        it = 2 * i2
        mom = stage(it, buf0, buf1, semI0, semI1, mom)
        mom = stage(it + 1, buf1, buf0, semI1, semI0, mom)
        return mom

    mom = lax.fori_loop(0, (NB - 1) // 2, body, mom0)
    mom = stage(jnp.int32(NB - 1), buf0, buf1, semI0, semI1, mom)
    wait_wb(buf0)
    for r in range(2, 8):
        for k in range(H // 16):
            momv[r, pl.ds(k * 16, 16)] = zv
    for k in range(H // 16):
        momv[0, pl.ds(k * 16, 16)] = mom[0][k]
        momv[1, pl.ds(k * 16, 16)] = mom[1][k]
    pltpu.sync_copy(momv, mom_hbm.at[wid])


def _k3_body(s_ref, st_ref):
    i = pl.program_id(0)
    pre1 = s_ref[...]

    @pl.when(i == 0)
    def _():
        st_ref[...] = jnp.zeros_like(st_ref)

    st_ref[0:1, :] += jnp.sum(pre1, axis=0, keepdims=True)
    st_ref[1:2, :] += jnp.sum(pre1 * pre1, axis=0, keepdims=True)


def _k4_body(s_ref, sc1_ref, sh1_ref, w2_ref, b2_ref,
             pre2_ref, st_ref):
    i = pl.program_id(0)
    h = jnp.maximum(s_ref[...] * sc1_ref[...] + sh1_ref[...], 0.0)
    pre2 = jnp.dot(h.astype(jnp.bfloat16), w2_ref[...].astype(jnp.bfloat16),
                   preferred_element_type=jnp.float32) + b2_ref[...]
    pre2_ref[...] = pre2.astype(jnp.bfloat16)

    @pl.when(i == 0)
    def _():
        st_ref[...] = jnp.zeros_like(st_ref)

    st_ref[0:1, :] += jnp.sum(pre2, axis=0, keepdims=True)
    st_ref[1:2, :] += jnp.sum(pre2 * pre2, axis=0, keepdims=True)


def _k5_body(pre2_ref, sc2_ref, sh2_ref, wse_ref, bse_ref, m_ref):
    msg = jnp.maximum(
        pre2_ref[...].astype(jnp.float32) * sc2_ref[...] + sh2_ref[...], 0.0)
    s = jnp.sum(msg * wse_ref[...], axis=1, keepdims=True) + bse_ref[0, 0]
    ew = jax.nn.sigmoid(s)
    m_ref[...] = msg * ew


def _k6_scatter_body(m_hbm, dst_hbm, out_hbm,
                     idx0, idx1, mb0, mb1, zbuf, acc,
                     semL0, semL1, semS0, semS1):
    cid = lax.axis_index("c")
    sid = lax.axis_index("s")
    wid = sid * NC + cid
    ebase = wid * EPW
    buf0 = (idx0, mb0, semL0, semS0)
    buf1 = (idx1, mb1, semL1, semS1)

    def zrow(j, c):
        for k in range(H // 16):
            zbuf[j, pl.ds(k * 16, 16)] = jnp.zeros((16,), jnp.float32)
        return c

    lax.fori_loop(0, ZB, zrow, 0)
    for cc in range((NCH + NS - 1) // NS):
        ch = sid + NS * cc

        @pl.when(ch < NCH)
        def _():
            pltpu.sync_copy(zbuf, acc.at[pl.ds(ch * ZB, ZB)])

    plsc.subcore_barrier()

    def fire_load(blk, b):
        idx, mb, semL, _ = b
        base = ebase + blk * BE
        pltpu.async_copy(dst_hbm.at[pl.ds(base, BE)], idx, semL)
        pltpu.async_copy(m_hbm.at[pl.ds(base, BE)], mb, semL)

    def wait_load(b):
        idx, mb, semL, _ = b
        pltpu.make_async_copy(dst_hbm.at[pl.ds(0, BE)], idx, semL).wait()
        pltpu.make_async_copy(m_hbm.at[pl.ds(0, BE)], mb, semL).wait()

    def fire_scatter(b):
        idx, mb, _, semS = b
        pltpu.async_copy(mb, acc.at[idx], semS, add=True)

    def wait_scatter(b):
        idx, mb, _, semS = b
        pltpu.make_async_copy(mb, acc.at[idx], semS).wait()

    def stage(it, cur, nxt):
        wait_load(cur)

        @pl.when(it >= 1)
        def _():
            wait_scatter(nxt)

        @pl.when(it + 1 < NB)
        def _():
            fire_load(it + 1, nxt)

        fire_scatter(cur)

    fire_load(0, buf0)

    def body(it, carry):
        @pl.when(it % 2 == 0)
        def _():
            stage(it, buf0, buf1)

        @pl.when(it % 2 == 1)
        def _():
            stage(it, buf1, buf0)

        return carry

    lax.fori_loop(0, NB, body, 0)
    wait_scatter(buf0 if (NB - 1) % 2 == 0 else buf1)
    plsc.subcore_barrier()
    for cc in range((NCH + NS - 1) // NS):
        ch = sid + NS * cc

        @pl.when(ch < NCH)
        def _():
            rows = pl.ds(ch * ZB, ZB)
            pltpu.sync_copy(acc.at[rows], out_hbm.at[cid, rows])


def _bn_scale_shift(s1, s2, n, g, be):
    mu = s1 / n
    var = s2 / n - mu * mu
    scale = g * lax.rsqrt(var + 1e-5)
    shift = be - mu * scale
    return scale, shift


@functools.cache
def _sc_kernels():
    mesh = plsc.VectorSubcoreMesh(core_axis_name="c", subcore_axis_name="s")
    k2 = functools.partial(
        pl.kernel,
        mesh=mesh,
        out_type=[
            jax.ShapeDtypeStruct((E, H), jnp.float32),
            jax.ShapeDtypeStruct((NW, 8, H), jnp.float32),
        ],
        scratch_types=[
            pltpu.VMEM((BE,), jnp.int32),
            pltpu.VMEM((BE,), jnp.int32),
            pltpu.VMEM((BE,), jnp.int32),
            pltpu.VMEM((BE,), jnp.int32),
            pltpu.VMEM((BE, DW), jnp.float32),
            pltpu.VMEM((BE, DW), jnp.float32),
            pltpu.VMEM((BE, DW), jnp.float32),
            pltpu.VMEM((BE, DW), jnp.float32),
            pltpu.VMEM((H,), jnp.float32),
            pltpu.VMEM((8, H), jnp.float32),
            pltpu.SemaphoreType.DMA,
            pltpu.SemaphoreType.DMA,
            pltpu.SemaphoreType.DMA,
            pltpu.SemaphoreType.DMA,
            pltpu.SemaphoreType.DMA,
            pltpu.SemaphoreType.DMA,
        ],
    )(_k2_gather_body)
    k6 = functools.partial(
        pl.kernel,
        mesh=mesh,
        out_type=jax.ShapeDtypeStruct((NC, N, H), jnp.float32),
        scratch_types=[
            pltpu.VMEM((BE,), jnp.int32),
            pltpu.VMEM((BE,), jnp.int32),
            pltpu.VMEM((BE, H), jnp.float32),
            pltpu.VMEM((BE, H), jnp.float32),
            pltpu.VMEM((ZB, H), jnp.float32),
            pltpu.VMEM_SHARED((N, H), jnp.float32),
            pltpu.SemaphoreType.DMA,
            pltpu.SemaphoreType.DMA,
            pltpu.SemaphoreType.DMA,
            pltpu.SemaphoreType.DMA,
        ],
    )(_k6_scatter_body)
    return k2, k6


def _k7_body(part_ref, feat_ref, wu1_ref, bu1_ref, gu1_ref, beu1_ref,
             wu2_ref, bu2_ref, gu2_ref, beu2_ref, out_ref):
    n = feat_ref.shape[0]
    feat = feat_ref[...]
    inp2 = part_ref[0] + part_ref[1] + feat
    pre1 = jnp.dot(inp2, wu1_ref[...], preferred_element_type=jnp.float32) + bu1_ref[...]
    s1 = jnp.sum(pre1, axis=0, keepdims=True)
    s2 = jnp.sum(pre1 * pre1, axis=0, keepdims=True)
    sc1, sh1 = _bn_scale_shift(s1, s2, n, gu1_ref[...], beu1_ref[...])
    hu = jnp.maximum(pre1 * sc1 + sh1, 0.0)
    pre2 = jnp.dot(hu, wu2_ref[...], preferred_element_type=jnp.float32) + bu2_ref[...]
    t1 = jnp.sum(pre2, axis=0, keepdims=True)
    t2 = jnp.sum(pre2 * pre2, axis=0, keepdims=True)
    sc2, sh2 = _bn_scale_shift(t1, t2, n, gu2_ref[...], beu2_ref[...])
    out_ref[...] = pre2 * sc2 + sh2 + feat


def kernel(x, feat, edge_index, W1, b1, g1, be1, W2, b2, g2, be2, Wse, bse,
           Wu1, bu1, gu1, beu1, Wu2, bu2, gu2, beu2):
    src = edge_index[0]
    dst = edge_index[1]
    w1a = W1[:H]
    w1b = W1[H:2 * H]
    w1c = W1[2 * H].reshape(1, H)
    t_tab, u_tab = pl.pallas_call(
        _k1_body,
        out_shape=[
            jax.ShapeDtypeStruct((N, DW), jnp.float32),
            jax.ShapeDtypeStruct((N, DW), jnp.float32),
        ],
    )(feat, x, w1a, w1b, b1.reshape(1, H))

    _k2_gather, _k6_scatter = _sc_kernels()
    s_arr, mom_arr = _k2_gather(t_tab, u_tab, w1c.reshape(H), src, dst)
    sc1, sh1 = _bn_scale_shift(jnp.sum(mom_arr[:, 0, :], axis=0).reshape(1, H),
                               jnp.sum(mom_arr[:, 1, :], axis=0).reshape(1, H),
                               E, g1.reshape(1, H), be1.reshape(1, H))

    GB4 = 4000
    pre2, st2 = pl.pallas_call(
        _k4_body,
        grid=(E // GB4,),
        in_specs=[
            pl.BlockSpec((GB4, H), lambda i: (i, 0)),
            pl.BlockSpec((1, H), lambda i: (0, 0)),
            pl.BlockSpec((1, H), lambda i: (0, 0)),
            pl.BlockSpec((H, H), lambda i: (0, 0)),
            pl.BlockSpec((1, H), lambda i: (0, 0)),
        ],
        out_specs=[
            pl.BlockSpec((GB4, H), lambda i: (i, 0)),
            pl.BlockSpec((8, H), lambda i: (0, 0)),
        ],
        out_shape=[
            jax.ShapeDtypeStruct((E, H), jnp.bfloat16),
            jax.ShapeDtypeStruct((8, H), jnp.float32),
        ],
    )(s_arr, sc1, sh1, W2, b2.reshape(1, H))
    sc2, sh2 = _bn_scale_shift(st2[0:1], st2[1:2], E, g2.reshape(1, H),
                               be2.reshape(1, H))

    GB5 = 4000
    m_arr = pl.pallas_call(
        _k5_body,
        grid=(E // GB5,),
        in_specs=[
            pl.BlockSpec((GB5, H), lambda i: (i, 0)),
            pl.BlockSpec((1, H), lambda i: (0, 0)),
            pl.BlockSpec((1, H), lambda i: (0, 0)),
            pl.BlockSpec((1, H), lambda i: (0, 0)),
            pl.BlockSpec((1, 1), lambda i: (0, 0)),
        ],
        out_specs=pl.BlockSpec((GB5, H), lambda i: (i, 0)),
        out_shape=jax.ShapeDtypeStruct((E, H), jnp.float32),
    )(pre2, sc2, sh2, Wse.reshape(1, H), bse.reshape(1, 1))

    partials = _k6_scatter(m_arr, dst)

    out = pl.pallas_call(
        _k7_body,
        out_shape=jax.ShapeDtypeStruct((N, H), jnp.float32),
    )(partials, feat, Wu1, bu1.reshape(1, H), gu1.reshape(1, H),
      beu1.reshape(1, H), Wu2, bu2.reshape(1, H), gu2.reshape(1, H),
      beu2.reshape(1, H))
    return out


# speedup vs baseline: 6.1721x; 1.0641x over previous
import functools

import jax
import jax.numpy as jnp
from jax import lax
from jax.experimental import pallas as pl
from jax.experimental.pallas import tpu as pltpu
from jax.experimental.pallas import tpu_sc as plsc

N = 10000
E = 320000
H = 128

NC = 2
NS = 16
NW = NC * NS
EPW = E // NW
BE = 80
NB = EPW // BE
ZB = 200
NCH = N // ZB
DW = 256

def _k1_body(feat_ref, x_ref, w1a_ref, w1b_ref, b1_ref, t_ref, u_ref):
    f = feat_ref[...]
    xx = x_ref[...]
    zpad = jnp.zeros((f.shape[0], DW - H - 3), jnp.float32)
    p = jnp.dot(f, w1a_ref[...], preferred_element_type=jnp.float32)
    q = jnp.dot(f, w1b_ref[...], preferred_element_type=jnp.float32) + b1_ref[...]
    t_ref[...] = jnp.concatenate([p, xx, zpad], axis=1)
    u_ref[...] = jnp.concatenate([q, -xx, zpad], axis=1)


def _k2_gather_body(t_hbm, u_hbm, w1c_hbm, src_hbm, dst_hbm, s_hbm, mom_hbm,
                    idx_s0, idx_d0, idx_s1, idx_d1,
                    bufP0, bufQ0, bufP1, bufQ1, w1cv, momv,
                    semG0, semG1, semW0, semW1, semI0, semI1):
    wid = lax.axis_index("s") * NC + lax.axis_index("c")
    ebase = wid * EPW
    buf0 = (idx_s0, idx_d0, bufP0, bufQ0, semG0, semW0)
    buf1 = (idx_s1, idx_d1, bufP1, bufQ1, semG1, semW1)
    pltpu.sync_copy(w1c_hbm, w1cv)
    w1c_regs = [w1cv[pl.ds(k * 16, 16)] for k in range(H // 16)]

    def fire_idx(blk, b, semI):
        idx_s, idx_d = b[0], b[1]
        base = ebase + blk * BE
        pltpu.async_copy(src_hbm.at[pl.ds(base, BE)], idx_s, semI)
        pltpu.async_copy(dst_hbm.at[pl.ds(base, BE)], idx_d, semI)

    def wait_idx(b, semI):
        idx_s, idx_d = b[0], b[1]
        pltpu.make_async_copy(src_hbm.at[pl.ds(0, BE)], idx_s, semI).wait()
        pltpu.make_async_copy(dst_hbm.at[pl.ds(0, BE)], idx_d, semI).wait()

    def fire_gather(b):
        idx_s, idx_d, bufP, bufQ, semG, _ = b
        pltpu.async_copy(t_hbm.at[idx_s], bufP, semG)
        pltpu.async_copy(u_hbm.at[idx_d], bufQ, semG)

    def wait_gather(b):
        idx_s, idx_d, bufP, bufQ, semG, _ = b
        pltpu.make_async_copy(t_hbm.at[idx_s], bufP, semG).wait()
        pltpu.make_async_copy(u_hbm.at[idx_d], bufQ, semG).wait()

    def fire_wb(blk, b):
        _, _, bufP, _, _, semW = b
        base = ebase + blk * BE
        pltpu.async_copy(bufP.at[:, pl.ds(0, H)], s_hbm.at[pl.ds(base, BE)], semW)

    def wait_wb(b):
        _, _, bufP, _, _, semW = b
        pltpu.make_async_copy(bufP.at[:, pl.ds(0, H)],
                              s_hbm.at[pl.ds(0, BE)], semW).wait()

    def compute(b, mom):
        _, _, bufP, bufQ, _, _ = b

        def row(j, mm):
            m1, m2 = mm
            vx = bufP[j, pl.ds(H, 16)] + bufQ[j, pl.ds(H, 16)]
            sq = vx[0] * vx[0] + vx[1] * vx[1] + vx[2] * vx[2]
            n1, n2 = [], []
            for k in range(H // 16):
                ksl = pl.ds(k * 16, 16)
                v = bufP[j, ksl] + bufQ[j, ksl] + sq * w1c_regs[k]
                bufP[j, ksl] = v
                n1.append(m1[k] + v)
                n2.append(m2[k] + v * v)
            return (tuple(n1), tuple(n2))

        return lax.fori_loop(0, BE, row, mom)

    def stage(it, cur, nxt, semIc, semIn, mom):
        wait_gather(cur)

        @pl.when(it + 2 < NB)
        def _():
            fire_idx(it + 2, cur, semIc)

        @pl.when(it >= 1)
        def _():
            wait_wb(nxt)

        @pl.when(it + 1 < NB)
        def _():
            wait_idx(nxt, semIn)
            fire_gather(nxt)

        mom = compute(cur, mom)
        fire_wb(it, cur)
        return mom

    pltpu.sync_copy(src_hbm.at[pl.ds(ebase, BE)], idx_s0)
    pltpu.sync_copy(dst_hbm.at[pl.ds(ebase, BE)], idx_d0)
    fire_gather(buf0)
    fire_idx(1, buf1, semI1)

    zv = jnp.zeros((16,), jnp.float32)
    mom0 = (tuple(zv for _ in range(H // 16)), tuple(zv for _ in range(H // 16)))

    def body(i2, mom):
        it = 2 * i2
        mom = stage(it, buf0, buf1, semI0, semI1, mom)
        mom = stage(it + 1, buf1, buf0, semI1, semI0, mom)
        return mom

    mom = lax.fori_loop(0, (NB - 1) // 2, body, mom0)
    mom = stage(jnp.int32(NB - 1), buf0, buf1, semI0, semI1, mom)
    wait_wb(buf0)
    for r in range(2, 8):
        for k in range(H // 16):
            momv[r, pl.ds(k * 16, 16)] = zv
    for k in range(H // 16):
        momv[0, pl.ds(k * 16, 16)] = mom[0][k]
        momv[1, pl.ds(k * 16, 16)] = mom[1][k]
    pltpu.sync_copy(momv, mom_hbm.at[wid])


def _k3_body(s_ref, st_ref):
    i = pl.program_id(0)
    pre1 = s_ref[...]

    @pl.when(i == 0)
    def _():
        st_ref[...] = jnp.zeros_like(st_ref)

    st_ref[0:1, :] += jnp.sum(pre1, axis=0, keepdims=True)
    st_ref[1:2, :] += jnp.sum(pre1 * pre1, axis=0, keepdims=True)


def _k4_body(s_ref, sc1_ref, sh1_ref, w2_ref, b2_ref,
             pre2_ref, st_ref):
    i = pl.program_id(0)
    h = jnp.maximum(s_ref[...] * sc1_ref[...] + sh1_ref[...], 0.0)
    pre2 = jnp.dot(h.astype(jnp.bfloat16), w2_ref[...].astype(jnp.bfloat16),
                   preferred_element_type=jnp.float32) + b2_ref[...]
    pre2_ref[...] = pre2.astype(jnp.bfloat16)

    @pl.when(i == 0)
    def _():
        st_ref[...] = jnp.zeros_like(st_ref)

    st_ref[0:1, :] += jnp.sum(pre2, axis=0, keepdims=True)
    st_ref[1:2, :] += jnp.sum(pre2 * pre2, axis=0, keepdims=True)


def _k5_body(pre2_ref, sc2_ref, sh2_ref, wse_ref, bse_ref, m_ref):
    msg = jnp.maximum(
        pre2_ref[...].astype(jnp.float32) * sc2_ref[...] + sh2_ref[...], 0.0)
    s = jnp.sum(msg * wse_ref[...], axis=1, keepdims=True) + bse_ref[0, 0]
    ew = jax.nn.sigmoid(s)
    m_ref[...] = msg * ew


def _k6_scatter_body(m_hbm, dst_hbm, out_hbm,
                     idx0, idx1, mb0, mb1, zbuf, acc,
                     semL0, semL1, semS0, semS1):
    cid = lax.axis_index("c")
    sid = lax.axis_index("s")
    wid = sid * NC + cid
    ebase = wid * EPW
    buf0 = (idx0, mb0, semL0, semS0)
    buf1 = (idx1, mb1, semL1, semS1)

    def zrow(j, c):
        for k in range(H // 16):
            zbuf[j, pl.ds(k * 16, 16)] = jnp.zeros((16,), jnp.float32)
        return c

    lax.fori_loop(0, ZB, zrow, 0)
    for cc in range((NCH + NS - 1) // NS):
        ch = sid + NS * cc

        @pl.when(ch < NCH)
        def _():
            pltpu.sync_copy(zbuf, acc.at[pl.ds(ch * ZB, ZB)])

    plsc.subcore_barrier()

    def fire_load(blk, b):
        idx, mb, semL, _ = b
        base = ebase + blk * BE
        pltpu.async_copy(dst_hbm.at[pl.ds(base, BE)], idx, semL)
        pltpu.async_copy(m_hbm.at[pl.ds(base, BE)], mb, semL)

    def wait_load(b):
        idx, mb, semL, _ = b
        pltpu.make_async_copy(dst_hbm.at[pl.ds(0, BE)], idx, semL).wait()
        pltpu.make_async_copy(m_hbm.at[pl.ds(0, BE)], mb, semL).wait()

    def fire_scatter(b):
        idx, mb, _, semS = b
        pltpu.async_copy(mb, acc.at[idx], semS, add=True)

    def wait_scatter(b):
        idx, mb, _, semS = b
        pltpu.make_async_copy(mb, acc.at[idx], semS).wait()

    def stage(it, cur, nxt):
        wait_load(cur)

        @pl.when(it >= 1)
        def _():
            wait_scatter(nxt)

        @pl.when(it + 1 < NB)
        def _():
            fire_load(it + 1, nxt)

        fire_scatter(cur)

    fire_load(0, buf0)

    def body(it, carry):
        @pl.when(it % 2 == 0)
        def _():
            stage(it, buf0, buf1)

        @pl.when(it % 2 == 1)
        def _():
            stage(it, buf1, buf0)

        return carry

    lax.fori_loop(0, NB, body, 0)
    wait_scatter(buf0 if (NB - 1) % 2 == 0 else buf1)
    plsc.subcore_barrier()
    for cc in range((NCH + NS - 1) // NS):
        ch = sid + NS * cc

        @pl.when(ch < NCH)
        def _():
            rows = pl.ds(ch * ZB, ZB)
            pltpu.sync_copy(acc.at[rows], out_hbm.at[cid, rows])


def _bn_scale_shift(s1, s2, n, g, be):
    mu = s1 / n
    var = s2 / n - mu * mu
    scale = g * lax.rsqrt(var + 1e-5)
    shift = be - mu * scale
    return scale, shift


@functools.cache
def _sc_kernels():
    mesh = plsc.VectorSubcoreMesh(core_axis_name="c", subcore_axis_name="s")
    k2 = functools.partial(
        pl.kernel,
        mesh=mesh,
        out_type=[
            jax.ShapeDtypeStruct((E, H), jnp.float32),
            jax.ShapeDtypeStruct((NW, 8, H), jnp.float32),
        ],
        scratch_types=[
            pltpu.VMEM((BE,), jnp.int32),
            pltpu.VMEM((BE,), jnp.int32),
            pltpu.VMEM((BE,), jnp.int32),
            pltpu.VMEM((BE,), jnp.int32),
            pltpu.VMEM((BE, DW), jnp.float32),
            pltpu.VMEM((BE, DW), jnp.float32),
            pltpu.VMEM((BE, DW), jnp.float32),
            pltpu.VMEM((BE, DW), jnp.float32),
            pltpu.VMEM((H,), jnp.float32),
            pltpu.VMEM((8, H), jnp.float32),
            pltpu.SemaphoreType.DMA,
            pltpu.SemaphoreType.DMA,
            pltpu.SemaphoreType.DMA,
            pltpu.SemaphoreType.DMA,
            pltpu.SemaphoreType.DMA,
            pltpu.SemaphoreType.DMA,
        ],
    )(_k2_gather_body)
    k6 = functools.partial(
        pl.kernel,
        mesh=mesh,
        out_type=jax.ShapeDtypeStruct((NC, N, H), jnp.float32),
        scratch_types=[
            pltpu.VMEM((BE,), jnp.int32),
            pltpu.VMEM((BE,), jnp.int32),
            pltpu.VMEM((BE, H), jnp.float32),
            pltpu.VMEM((BE, H), jnp.float32),
            pltpu.VMEM((ZB, H), jnp.float32),
            pltpu.VMEM_SHARED((N, H), jnp.float32),
            pltpu.SemaphoreType.DMA,
            pltpu.SemaphoreType.DMA,
            pltpu.SemaphoreType.DMA,
            pltpu.SemaphoreType.DMA,
        ],
    )(_k6_scatter_body)
    return k2, k6


def _k7_body(part_ref, feat_ref, wu1_ref, bu1_ref, gu1_ref, beu1_ref,
             wu2_ref, bu2_ref, gu2_ref, beu2_ref, out_ref):
    n = feat_ref.shape[0]
    feat = feat_ref[...]
    inp2 = part_ref[0] + part_ref[1] + feat
    pre1 = jnp.dot(inp2, wu1_ref[...], preferred_element_type=jnp.float32) + bu1_ref[...]
    s1 = jnp.sum(pre1, axis=0, keepdims=True)
    s2 = jnp.sum(pre1 * pre1, axis=0, keepdims=True)
    sc1, sh1 = _bn_scale_shift(s1, s2, n, gu1_ref[...], beu1_ref[...])
    hu = jnp.maximum(pre1 * sc1 + sh1, 0.0)
    pre2 = jnp.dot(hu, wu2_ref[...], preferred_element_type=jnp.float32) + bu2_ref[...]
    t1 = jnp.sum(pre2, axis=0, keepdims=True)
    t2 = jnp.sum(pre2 * pre2, axis=0, keepdims=True)
    sc2, sh2 = _bn_scale_shift(t1, t2, n, gu2_ref[...], beu2_ref[...])
    out_ref[...] = pre2 * sc2 + sh2 + feat


def kernel(x, feat, edge_index, W1, b1, g1, be1, W2, b2, g2, be2, Wse, bse,
           Wu1, bu1, gu1, beu1, Wu2, bu2, gu2, beu2):
    src = edge_index[0]
    dst = edge_index[1]
    w1a = W1[:H]
    w1b = W1[H:2 * H]
    w1c = W1[2 * H].reshape(1, H)
    t_tab, u_tab = pl.pallas_call(
        _k1_body,
        out_shape=[
            jax.ShapeDtypeStruct((N, DW), jnp.float32),
            jax.ShapeDtypeStruct((N, DW), jnp.float32),
        ],
    )(feat, x, w1a, w1b, b1.reshape(1, H))

    _k2_gather, _k6_scatter = _sc_kernels()
    s_arr, mom_arr = _k2_gather(t_tab, u_tab, w1c.reshape(H), src, dst)
    sc1, sh1 = _bn_scale_shift(jnp.sum(mom_arr[:, 0, :], axis=0).reshape(1, H),
                               jnp.sum(mom_arr[:, 1, :], axis=0).reshape(1, H),
                               E, g1.reshape(1, H), be1.reshape(1, H))

    GB4 = 8000
    pre2, st2 = pl.pallas_call(
        _k4_body,
        grid=(E // GB4,),
        in_specs=[
            pl.BlockSpec((GB4, H), lambda i: (i, 0)),
            pl.BlockSpec((1, H), lambda i: (0, 0)),
            pl.BlockSpec((1, H), lambda i: (0, 0)),
            pl.BlockSpec((H, H), lambda i: (0, 0)),
            pl.BlockSpec((1, H), lambda i: (0, 0)),
        ],
        out_specs=[
            pl.BlockSpec((GB4, H), lambda i: (i, 0)),
            pl.BlockSpec((8, H), lambda i: (0, 0)),
        ],
        out_shape=[
            jax.ShapeDtypeStruct((E, H), jnp.bfloat16),
            jax.ShapeDtypeStruct((8, H), jnp.float32),
        ],
    )(s_arr, sc1, sh1, W2, b2.reshape(1, H))
    sc2, sh2 = _bn_scale_shift(st2[0:1], st2[1:2], E, g2.reshape(1, H),
                               be2.reshape(1, H))

    GB5 = 8000
    m_arr = pl.pallas_call(
        _k5_body,
        grid=(E // GB5,),
        in_specs=[
            pl.BlockSpec((GB5, H), lambda i: (i, 0)),
            pl.BlockSpec((1, H), lambda i: (0, 0)),
            pl.BlockSpec((1, H), lambda i: (0, 0)),
            pl.BlockSpec((1, H), lambda i: (0, 0)),
            pl.BlockSpec((1, 1), lambda i: (0, 0)),
        ],
        out_specs=pl.BlockSpec((GB5, H), lambda i: (i, 0)),
        out_shape=jax.ShapeDtypeStruct((E, H), jnp.float32),
    )(pre2, sc2, sh2, Wse.reshape(1, H), bse.reshape(1, 1))

    partials = _k6_scatter(m_arr, dst)

    out = pl.pallas_call(
        _k7_body,
        out_shape=jax.ShapeDtypeStruct((N, H), jnp.float32),
    )(partials, feat, Wu1, bu1.reshape(1, H), gu1.reshape(1, H),
      beu1.reshape(1, H), Wu2, bu2.reshape(1, H), gu2.reshape(1, H),
      beu2.reshape(1, H))
    return out
